# baseline (speedup 1.0000x reference)
import functools

import jax
import jax.numpy as jnp
from jax import lax
from jax.experimental import pallas as pl
from jax.experimental.pallas import tpu as pltpu
from jax.experimental.pallas import tpu_sc as plsc

M = 4096
N = 4096
NW = 32
R = 8
ROWS_PER_W = M // NW


def _sc_body(x_hbm, m_hbm, out_hbm, xv, mv, ov):
    wid = lax.axis_index("s") * 2 + lax.axis_index("c")
    base0 = wid * ROWS_PER_W

    def chunk_body(k, _):
        b = base0 + k * R
        pltpu.sync_copy(x_hbm.at[pl.ds(b, R)], xv)
        pltpu.sync_copy(m_hbm.at[pl.ds(b, R)], mv)
        for r in range(R):

            def col_body(c, carry):
                off = c * 16
                v = xv[r, pl.ds(off, 16)]
                mf = mv[r, pl.ds(off, 16)].astype(jnp.float32)
                masked = v * mf
                cs = plsc.cumsum(masked)
                ov[r, pl.ds(off, 16)] = cs + carry
                return carry + jnp.sum(masked)

            lax.fori_loop(0, N // 16, col_body, jnp.float32(0.0))
        pltpu.sync_copy(ov, out_hbm.at[pl.ds(b, R)])
        return 0

    lax.fori_loop(0, ROWS_PER_W // R, chunk_body, 0)


@jax.jit
def kernel(x, mask):
    m32 = mask.astype(jnp.int32)
    sc = pl.kernel(
        _sc_body,
        mesh=plsc.VectorSubcoreMesh(core_axis_name="c", subcore_axis_name="s"),
        out_type=jax.ShapeDtypeStruct((M, N), jnp.float32),
        scratch_types=[
            pltpu.VMEM((R, N), jnp.float32),
            pltpu.VMEM((R, N), jnp.int32),
            pltpu.VMEM((R, N), jnp.float32),
        ],
        compiler_params=pltpu.CompilerParams(needs_layout_passes=False),
    )
    return sc(x, m32)


# speedup vs baseline: 4.3046x; 4.3046x over previous
import jax
import jax.numpy as jnp
from jax.experimental import pallas as pl
from jax.experimental.pallas import tpu as pltpu

M = 4096
N = 4096
BM = 4096
BN = 512


def _cumsum_kernel(x_ref, mask_ref, out_ref, carry_ref):
    j = pl.program_id(1)

    @pl.when(j == 0)
    def _():
        carry_ref[...] = jnp.zeros_like(carry_ref)

    masked = x_ref[...].astype(jnp.bfloat16) * mask_ref[...].astype(jnp.bfloat16)

    rows = jax.lax.broadcasted_iota(jnp.int32, (BN, BN), 0)
    cols = jax.lax.broadcasted_iota(jnp.int32, (BN, BN), 1)
    tri = (rows <= cols).astype(jnp.bfloat16)

    csum = jax.lax.dot(masked, tri, preferred_element_type=jnp.float32)

    carry = carry_ref[:, :1]
    out_ref[...] = csum + carry
    carry_ref[...] = jnp.broadcast_to(carry + csum[:, -1:], carry_ref.shape)


@jax.jit
def kernel(x, mask):
    mask = mask.astype(jnp.int4)
    grid = (M // BM, N // BN)
    return pl.pallas_call(
        _cumsum_kernel,
        grid=grid,
        in_specs=[
            pl.BlockSpec((BM, BN), lambda i, j: (i, j)),
            pl.BlockSpec((BM, BN), lambda i, j: (i, j)),
        ],
        out_specs=pl.BlockSpec((BM, BN), lambda i, j: (i, j)),
        out_shape=jax.ShapeDtypeStruct((M, N), jnp.float32),
        scratch_shapes=[pltpu.VMEM((BM, 128), jnp.float32)],
        compiler_params=pltpu.CompilerParams(
            dimension_semantics=("arbitrary", "arbitrary"),
            allow_input_fusion=(False, True),
        ),
    )(x, mask)
